# baseline (speedup 1.0000x reference)
import jax
import jax.numpy as jnp
from jax import lax
from jax.experimental import pallas as pl
from jax.experimental.pallas import tpu as pltpu
from jax.experimental.pallas import tpu_sc as plsc

_N, _D, _E = 2048, 1024, 8
_TN = 1024
_TB = 128
_TILES = 31 + _E
_ROWS = _TILES * _TB
_XG_ROWS = (_TILES + 1) * _TB
_DUMP = _XG_ROWS - 1
_NP = 2 * _N
_PPW = _NP // 16
_CH = 32
_TPW = _N // 32
_SL = _D // 128


def _leaky(v):
    return jnp.where(v >= 0, v, 0.01 * v)


def _iota16():
    return lax.broadcasted_iota(jnp.int32, (16,), 0)


def _router_body(x_ref, W1_ref, b1_ref, W2_ref, b2_ref, eidx_ref, probs_ref):
    x = x_ref[...]
    h = jnp.dot(x, W1_ref[...], preferred_element_type=jnp.float32)
    h = _leaky(h + b1_ref[...])
    logits = jnp.dot(h, W2_ref[...], preferred_element_type=jnp.float32)
    logits = logits + b2_ref[...]
    eidx = lax.broadcasted_iota(jnp.int32, logits.shape, 1)
    m1 = jnp.max(logits, axis=1, keepdims=True)
    i1 = jnp.min(jnp.where(logits == m1, eidx, _E), axis=1, keepdims=True)
    masked = jnp.where(eidx == i1, -jnp.inf, logits)
    m2 = jnp.max(masked, axis=1, keepdims=True)
    i2 = jnp.min(jnp.where(masked == m2, eidx, _E), axis=1, keepdims=True)
    p2 = 1.0 / (1.0 + jnp.exp(m1 - m2))
    p1 = 1.0 - p2
    eidx_ref[...] = jnp.concatenate([i1, i2], axis=1)
    probs_ref[...] = jnp.concatenate([p1, p2], axis=1)


def _router(x, W1, b1, W2, b2):
    nt = _N // _TN
    return pl.pallas_call(
        _router_body,
        grid=(nt,),
        in_specs=[
            pl.BlockSpec((_TN, _D), lambda n: (n, 0)),
            pl.BlockSpec((_D, _D), lambda n: (0, 0)),
            pl.BlockSpec((1, _D), lambda n: (0, 0)),
            pl.BlockSpec((_D, _E), lambda n: (0, 0)),
            pl.BlockSpec((1, _E), lambda n: (0, 0)),
        ],
        out_specs=[
            pl.BlockSpec((_TN, 2), lambda n: (n, 0)),
            pl.BlockSpec((_TN, 2), lambda n: (n, 0)),
        ],
        out_shape=[
            jax.ShapeDtypeStruct((_N, 2), jnp.int32),
            jax.ShapeDtypeStruct((_N, 2), jnp.float32),
        ],
        compiler_params=pltpu.CompilerParams(
            dimension_semantics=("arbitrary",),
        ),
    )(x, W1, b1.reshape(1, _D), W2, b2.reshape(1, _E))


def _vsum(v, tmpbuf):
    iota = _iota16()
    for k in (1, 2, 4, 8):
        tmpbuf[...] = v
        v = v + plsc.load_gather(tmpbuf, [jnp.bitwise_xor(iota, k)])
    return v


def _dispatch_body(ep_hbm, x_hbm, xg_hbm, posp1_hbm, eot_hbm,
                   ebuf, cnt2d, pre2d,
                   posbuf, tl0, tl1, tl2, tl3, pl0, pl1, pl2, pl3,
                   tokidx, dstidx, rowbuf, eotbuf, tmpbuf, gsem, ssem):
    toklist = [tl0, tl1, tl2, tl3]
    plist = [pl0, pl1, pl2, pl3]
    c = lax.axis_index("c")
    s = lax.axis_index("s")
    iota = _iota16()
    zero16 = jnp.zeros((16,), jnp.int32)
    ones16 = jnp.full((16,), 1, jnp.int32)

    pltpu.sync_copy(ep_hbm, ebuf)

    for j in range(16):
        cnt2d[pl.ds(j * 16, 16)] = zero16
        pre2d[pl.ds(j * 16, 16)] = zero16
    for g in range(_NP // 16):
        ev = ebuf[pl.ds(g * 16, 16)]
        idx = ev * 16 + iota
        plsc.addupdate_scatter(cnt2d, [idx], ones16)
        inpre = jnp.where(jnp.int32(g * 16) < s * _PPW, ones16, zero16)
        plsc.addupdate_scatter(pre2d, [idx], inpre)

    tot = zero16
    pre = zero16
    for e in range(_E):
        te = _vsum(cnt2d[pl.ds(e * 16, 16)], tmpbuf)
        pe = _vsum(pre2d[pl.ds(e * 16, 16)], tmpbuf)
        tot = jnp.where(iota == e, te, tot)
        pre = jnp.where(iota == e, pe, pre)
    padded = jnp.bitwise_and(tot + (_TB - 1), jnp.int32(-_TB))
    run = jnp.int32(0)
    basev = zero16
    for e in range(_E):
        basev = jnp.where(iota == e, run, basev)
        run = run + padded[e]
    stv = basev + pre

    @pl.when(jnp.logical_and(c == 0, s == 0))
    def _write_eot():
        btv = lax.shift_right_arithmetic(basev, 7)
        for jv in range(4):
            tid = iota + jv * 16
            acc = jnp.zeros((16,), jnp.int32)
            for e in range(_E):
                acc = acc + jnp.where(tid >= btv[e], 1, 0)
            eotbuf[pl.ds(jv * 16, 16)] = jnp.maximum(acc - 1, 0)
        pltpu.sync_copy(eotbuf, eot_hbm)

    for j in range(_PPW // 16):
        posbuf[pl.ds(j * 16, 16)] = zero16
    for i in range(4):
        for j in range(_PPW // 16 + 1):
            toklist[i][pl.ds(j * 16, 16)] = zero16

    tmpbuf[...] = stv
    es = [zero16 + (4 * c + i) for i in range(4)]
    sts = [plsc.load_gather(tmpbuf, [es[i]]) for i in range(4)]
    offs = [jnp.int32(0)] * 4
    for j in range(_PPW // 16):
        ev = ebuf[pl.ds(s * _PPW + j * 16, 16)]
        lidx = iota + j * 16
        tokv = lax.shift_right_arithmetic(lidx + s * _PPW, 1)
        for i in range(4):
            m = ev == es[i]
            plsc.store_compressed(toklist[i].at[pl.ds(offs[i], 16)], tokv,
                                  mask=m)
            plsc.store_compressed(plist[i].at[pl.ds(offs[i], 16)], lidx,
                                  mask=m)
            offs[i] = offs[i] + _vsum(jnp.where(m, 1, 0), tmpbuf)[0]

    for i in range(4):
        for j in range(_PPW // 16):
            @pl.when(j * 16 < offs[i])
            def _inv(i=i, j=j):
                pv = plist[i][pl.ds(j * 16, 16)]
                valid = (iota + j * 16) < offs[i]
                posvals = sts[i] + iota + (j * 16 + 1)
                plsc.store_scatter(posbuf, [pv], posvals, mask=valid)

    pltpu.sync_copy(posbuf, posp1_hbm.at[c, pl.ds(s * _PPW, _PPW)])

    for i in range(4):
        for ch in range(_PPW // _CH):
            @pl.when(ch * _CH < offs[i])
            def _move(i=i, ch=ch):
                for j in range(_CH // 16):
                    lane = ch * _CH + j * 16
                    tokidx[pl.ds(j * 16, 16)] = toklist[i][pl.ds(lane, 16)]
                    dval = sts[i] + lane + iota
                    valid = (lane + iota) < offs[i]
                    dstidx[pl.ds(j * 16, 16)] = jnp.where(valid, dval, _DUMP)
                pltpu.async_copy(x_hbm.at[tokidx], rowbuf, gsem).wait()
                pltpu.async_copy(rowbuf, xg_hbm.at[dstidx], ssem).wait()


def _dispatch(ep_flat, x3):
    mesh = plsc.VectorSubcoreMesh(core_axis_name="c", subcore_axis_name="s",
                                  num_cores=2, num_subcores=16)
    f = pl.kernel(
        _dispatch_body,
        out_type=[
            jax.ShapeDtypeStruct((_XG_ROWS, _SL, 128), jnp.float32),
            jax.ShapeDtypeStruct((2, _NP), jnp.int32),
            jax.ShapeDtypeStruct((64,), jnp.int32),
        ],
        mesh=mesh,
        compiler_params=pltpu.CompilerParams(needs_layout_passes=False),
        scratch_types=[
            pltpu.VMEM((_NP,), jnp.int32),
            pltpu.VMEM((256,), jnp.int32),
            pltpu.VMEM((256,), jnp.int32),
            pltpu.VMEM((_PPW,), jnp.int32),
        ] + [pltpu.VMEM((_PPW + 16,), jnp.int32)] * 8 + [
            pltpu.VMEM((_CH,), jnp.int32),
            pltpu.VMEM((_CH,), jnp.int32),
            pltpu.VMEM((_CH, _SL, 128), jnp.float32),
            pltpu.VMEM((64,), jnp.int32),
            pltpu.VMEM((16,), jnp.int32),
            pltpu.SemaphoreType.DMA,
            pltpu.SemaphoreType.DMA,
        ],
    )
    return f(ep_flat, x3)


def _mm_body(eot_ref, xg_ref, We_ref, y_ref):
    y_ref[...] = jnp.dot(xg_ref[...].astype(jnp.bfloat16), We_ref[0],
                         preferred_element_type=jnp.float32)


def _expert_mm(eot, xg, Web):
    return pl.pallas_call(
        _mm_body,
        grid_spec=pltpu.PrefetchScalarGridSpec(
            num_scalar_prefetch=1,
            grid=(_TILES,),
            in_specs=[
                pl.BlockSpec((_TB, _D), lambda i, eot: (i, 0)),
                pl.BlockSpec((1, _D, _D), lambda i, eot: (eot[i], 0, 0)),
            ],
            out_specs=pl.BlockSpec((_TB, _D), lambda i, eot: (i, 0)),
        ),
        out_shape=jax.ShapeDtypeStruct((_ROWS, _D), jnp.float32),
        compiler_params=pltpu.CompilerParams(
            dimension_semantics=("arbitrary",),
        ),
    )(eot, xg, Web)


def _combine_body(y_hbm, pr_hbm, ep_hbm, posp1_hbm, be_hbm, out_hbm,
                  pbuf, ebuf, p0buf, p1buf, posbuf, posidx, betab, ybuf,
                  outbuf, gsem):
    c = lax.axis_index("c")
    s = lax.axis_index("s")
    w = s * 2 + c
    pbase = w * (2 * _TPW)

    pltpu.sync_copy(pr_hbm.at[pl.ds(pbase, 2 * _TPW)], pbuf)
    pltpu.sync_copy(ep_hbm.at[pl.ds(pbase, 2 * _TPW)], ebuf)
    pltpu.sync_copy(posp1_hbm.at[0, pl.ds(pbase, 2 * _TPW)], p0buf)
    pltpu.sync_copy(posp1_hbm.at[1, pl.ds(pbase, 2 * _TPW)], p1buf)
    pltpu.sync_copy(be_hbm, betab)
    for j in range((2 * _TPW) // 16):
        posbuf[pl.ds(j * 16, 16)] = (p0buf[pl.ds(j * 16, 16)]
                                     + p1buf[pl.ds(j * 16, 16)] - 1)

    ntok_ch = _CH // 2
    for ch in range(_TPW // ntok_ch):
        for j in range(_CH // 16):
            posidx[pl.ds(j * 16, 16)] = posbuf[pl.ds(ch * _CH + j * 16, 16)]
        pltpu.async_copy(y_hbm.at[posidx], ybuf, gsem).wait()
        pv = [pbuf[pl.ds(ch * _CH + j * 16, 16)] for j in range(2)]
        ev = [ebuf[pl.ds(ch * _CH + j * 16, 16)] for j in range(2)]
        for t in range(ntok_ch):
            q = 2 * t
            pa1 = pv[q // 16][q % 16]
            pa2 = pv[q // 16][q % 16 + 1]
            e1 = ev[q // 16][q % 16]
            e2 = ev[q // 16][q % 16 + 1]

            def _col(j, _, t=t, pa1=pa1, pa2=pa2, e1=e1, e2=e2):
                a = j // 8
                b = (j % 8) * 16
                ya = ybuf[2 * t, a, pl.ds(b, 16)] + betab[e1, a, pl.ds(b, 16)]
                yb = (ybuf[2 * t + 1, a, pl.ds(b, 16)]
                      + betab[e2, a, pl.ds(b, 16)])
                o = pa1 * ya + pa2 * yb
                outbuf[t, a, pl.ds(b, 16)] = _leaky(o)
                return 0

            lax.fori_loop(0, _D // 16, _col, 0, unroll=4)
        pltpu.sync_copy(outbuf,
                        out_hbm.at[pl.ds(w * _TPW + ch * ntok_ch, ntok_ch)])


def _combine(y3, probs_flat, ep_flat, posp1, be3):
    mesh = plsc.VectorSubcoreMesh(core_axis_name="c", subcore_axis_name="s",
                                  num_cores=2, num_subcores=16)
    ntok_ch = _CH // 2
    f = pl.kernel(
        _combine_body,
        out_type=jax.ShapeDtypeStruct((_N, _SL, 128), jnp.float32),
        mesh=mesh,
        compiler_params=pltpu.CompilerParams(needs_layout_passes=False),
        scratch_types=[
            pltpu.VMEM((2 * _TPW,), jnp.float32),
            pltpu.VMEM((2 * _TPW,), jnp.int32),
            pltpu.VMEM((2 * _TPW,), jnp.int32),
            pltpu.VMEM((2 * _TPW,), jnp.int32),
            pltpu.VMEM((2 * _TPW,), jnp.int32),
            pltpu.VMEM((_CH,), jnp.int32),
            pltpu.VMEM((_E, _SL, 128), jnp.float32),
            pltpu.VMEM((_CH, _SL, 128), jnp.float32),
            pltpu.VMEM((ntok_ch, _SL, 128), jnp.float32),
            pltpu.SemaphoreType.DMA,
        ],
    )
    return f(y3, probs_flat, ep_flat, posp1, be3)


def kernel(x, W1, b1, W2, b2, We, be):
    eidx, probs = _router(x, W1, b1, W2, b2)
    ep_flat = eidx.reshape(_NP)
    x3 = x.reshape(_N, _SL, 128)
    xg3, posp1, eot = _dispatch(ep_flat, x3)
    Web = We.astype(jnp.bfloat16)
    y = _expert_mm(eot, xg3.reshape(_XG_ROWS, _D)[:_ROWS], Web)
    out3 = _combine(y.reshape(_ROWS, _SL, 128), probs.reshape(_NP), ep_flat,
                    posp1, be.reshape(_E, _SL, 128))
    return out3.reshape(_N, _D)


# speedup vs baseline: 5.3351x; 5.3351x over previous
import jax
import jax.numpy as jnp
from jax.experimental import pallas as pl
from jax.experimental.pallas import tpu as pltpu

_N, _D, _E = 2048, 1024, 8
_TN = 1024


def _leaky(v):
    return jnp.where(v >= 0, v, 0.01 * v)


def _body(x_ref, W1_ref, b1_ref, W2_ref, b2_ref, Wcat_ref, be_ref, out_ref):
    x = x_ref[...]
    h = jnp.dot(x, W1_ref[...], preferred_element_type=jnp.float32)
    h = _leaky(h + b1_ref[...])
    logits = jnp.dot(h, W2_ref[...], preferred_element_type=jnp.float32)
    logits = logits + b2_ref[...]
    eidx = jax.lax.broadcasted_iota(jnp.int32, logits.shape, 1)
    m1 = jnp.max(logits, axis=1, keepdims=True)
    i1 = jnp.min(jnp.where(logits == m1, eidx, _E), axis=1, keepdims=True)
    masked = jnp.where(eidx == i1, -jnp.inf, logits)
    m2 = jnp.max(masked, axis=1, keepdims=True)
    i2 = jnp.min(jnp.where(masked == m2, eidx, _E), axis=1, keepdims=True)
    p2 = 1.0 / (1.0 + jnp.exp(m1 - m2))
    p1 = 1.0 - p2
    c = jnp.where(eidx == i1, p1, 0.0) + jnp.where(eidx == i2, p2, 0.0)

    xcat = jnp.concatenate(
        [(c[:, e:e + 1] * x).astype(jnp.bfloat16) for e in range(_E)], axis=1)
    y = jnp.dot(xcat, Wcat_ref[...], preferred_element_type=jnp.float32)
    bias = jnp.dot(c, be_ref[...], preferred_element_type=jnp.float32)
    out_ref[...] = _leaky(y + bias)


def kernel(x, W1, b1, W2, b2, We, be):
    nt = _N // _TN
    Wcat = We.astype(jnp.bfloat16).reshape(_E * _D, _D)
    out = pl.pallas_call(
        _body,
        grid=(nt,),
        in_specs=[
            pl.BlockSpec((_TN, _D), lambda n: (n, 0)),
            pl.BlockSpec((_D, _D), lambda n: (0, 0)),
            pl.BlockSpec((1, _D), lambda n: (0, 0)),
            pl.BlockSpec((_D, _E), lambda n: (0, 0)),
            pl.BlockSpec((1, _E), lambda n: (0, 0)),
            pl.BlockSpec((_E * _D, _D), lambda n: (0, 0)),
            pl.BlockSpec((_E, _D), lambda n: (0, 0)),
        ],
        out_specs=pl.BlockSpec((_TN, _D), lambda n: (n, 0)),
        out_shape=jax.ShapeDtypeStruct((_N, _D), jnp.float32),
        compiler_params=pltpu.CompilerParams(
            dimension_semantics=("arbitrary",),
        ),
    )(x, W1, b1.reshape(1, _D), W2, b2.reshape(1, _E), Wcat, be)
    return out
